# baseline (speedup 1.0000x reference)
import functools

import jax
import jax.numpy as jnp
from jax import lax
from jax.experimental import pallas as pl
from jax.experimental.pallas import tpu as pltpu
from jax.experimental.pallas import tpu_sc as plsc

N_EMB = 1024
D = 64
B_IN = 16
HW_IN = 1024
N_TOK = B_IN * HW_IN
BLK = 2048
GRID = N_TOK // BLK

NC, NS = 2, 16
NW = NC * NS
TOK_PER_W = N_TOK // NW
IDX_CHUNK = 128
N_CHUNK = TOK_PER_W // IDX_CHUNK


def _tc_body(x_ref, cb_ref, idx_ref, loss_ref, ppl_ref, hist_ref, acc_ref):
    i = pl.program_id(0)
    x = x_ref[...]
    cb = cb_ref[...]
    xn = jnp.sum(x * x, axis=1, keepdims=True)
    cn = jnp.sum(cb * cb, axis=0, keepdims=True)
    mm2 = jnp.dot(x * -2.0, cb, preferred_element_type=jnp.float32)
    d = (xn + cn) + mm2
    m = jnp.min(d, axis=1, keepdims=True)
    iota = lax.broadcasted_iota(jnp.int32, (BLK, N_EMB), 1)
    idx = jnp.min(jnp.where(d == m, iota, N_EMB), axis=1, keepdims=True)
    idx_ref[...] = idx

    @pl.when(i == 0)
    def _init():
        hist_ref[...] = jnp.zeros_like(hist_ref)
        acc_ref[0, 0] = 0.0

    hist_ref[...] += jnp.sum(jnp.where(iota == idx, 1.0, 0.0),
                             axis=0, keepdims=True)
    acc_ref[0, 0] += jnp.sum(m)

    loss_ref[...] = jnp.full((1, 1), acc_ref[0, 0] * (1.0 / (N_TOK * D)),
                             jnp.float32)

    @pl.when(i == GRID - 1)
    def _finish():
        p = hist_ref[...] * (1.0 / N_TOK)
        ent = jnp.sum(p * jnp.log(p + 1e-10))
        ppl_ref[...] = jnp.full((1, 1), jnp.exp(-ent), jnp.float32)

    @pl.when(i < GRID - 1)
    def _keep():
        ppl_ref[...] = jnp.zeros((1, 1), jnp.float32)


def _sc_body(idx_hbm, table_hbm, out_hbm, idx_v, rows_v, sem):
    wid = lax.axis_index("s") * NC + lax.axis_index("c")
    pltpu.sync_copy(idx_hbm.at[pl.ds(wid * N_CHUNK, N_CHUNK)], idx_v)
    copies = []
    for j in range(N_CHUNK):
        copies.append(pltpu.async_copy(
            table_hbm.at[idx_v.at[j]],
            rows_v.at[pl.ds(j * IDX_CHUNK, IDX_CHUNK)],
            sem))
    for c in copies:
        c.wait()
    pltpu.sync_copy(rows_v, out_hbm.at[pl.ds(wid * TOK_PER_W, TOK_PER_W)])


@functools.cache
def _sc_gather():
    return pl.kernel(
        _sc_body,
        out_type=jax.ShapeDtypeStruct((N_TOK, D), jnp.float32),
        mesh=plsc.VectorSubcoreMesh(core_axis_name="c", subcore_axis_name="s",
                                    num_cores=NC, num_subcores=NS),
        scratch_types=[
            pltpu.VMEM((N_CHUNK, IDX_CHUNK), jnp.int32),
            pltpu.VMEM((TOK_PER_W, D), jnp.float32),
            pltpu.SemaphoreType.DMA,
        ],
        compiler_params=pltpu.CompilerParams(use_tc_tiling_on_sc=False),
    )


def kernel(inputs, codebook):
    flat = inputs.reshape(N_TOK, D)
    idx, loss, ppl = pl.pallas_call(
        _tc_body,
        grid=(GRID,),
        in_specs=[
            pl.BlockSpec((BLK, D), lambda i: (i, 0)),
            pl.BlockSpec((D, N_EMB), lambda i: (0, 0)),
        ],
        out_specs=[
            pl.BlockSpec((BLK, 1), lambda i: (i, 0)),
            pl.BlockSpec((1, 1), lambda i: (0, 0)),
            pl.BlockSpec((1, 1), lambda i: (0, 0)),
        ],
        out_shape=[
            jax.ShapeDtypeStruct((N_TOK, 1), jnp.int32),
            jax.ShapeDtypeStruct((1, 1), jnp.float32),
            jax.ShapeDtypeStruct((1, 1), jnp.float32),
        ],
        scratch_shapes=[
            pltpu.VMEM((1, N_EMB), jnp.float32),
            pltpu.SMEM((1, 1), jnp.float32),
        ],
    )(flat, codebook)

    idx2 = idx.reshape(NW * N_CHUNK, IDX_CHUNK)
    qflat = _sc_gather()(idx2, codebook.T)
    quantized = qflat.reshape(B_IN, HW_IN, D)
    return quantized, loss[0, 0], ppl[0, 0]


# speedup vs baseline: 1.0602x; 1.0602x over previous
import functools

import jax
import jax.numpy as jnp
from jax import lax
from jax.experimental import pallas as pl
from jax.experimental.pallas import tpu as pltpu
from jax.experimental.pallas import tpu_sc as plsc

N_EMB = 1024
D = 64
B_IN = 16
HW_IN = 1024
N_TOK = B_IN * HW_IN
BLK = 2048
GRID = N_TOK // BLK

NC, NS = 2, 16
NW = NC * NS
TOK_PER_W = N_TOK // NW
IDX_CHUNK = 128
N_CHUNK = TOK_PER_W // IDX_CHUNK


def _tc_body(x_ref, cb_ref, idx_ref, loss_ref, acc_ref):
    i = pl.program_id(0)
    x = x_ref[...]
    cb = cb_ref[...]
    xn = jnp.sum(x * x, axis=1, keepdims=True)
    cn = jnp.sum(cb * cb, axis=0, keepdims=True)
    mm2 = jnp.dot(x * -2.0, cb, preferred_element_type=jnp.float32)
    d = (xn + cn) + mm2
    m = jnp.min(d, axis=1, keepdims=True)
    iota = lax.broadcasted_iota(jnp.int32, (BLK, N_EMB), 1)
    idx = jnp.min(jnp.where(d == m, iota, N_EMB), axis=1, keepdims=True)
    idx_ref[...] = idx

    @pl.when(i == 0)
    def _init():
        acc_ref[0, 0] = 0.0

    acc_ref[0, 0] += jnp.sum(m)
    loss_ref[...] = jnp.full((1, 1), acc_ref[0, 0] * (1.0 / (N_TOK * D)),
                             jnp.float32)


def _sc_body(idx_hbm, table_hbm, out_hbm, hist_hbm, idx_v, rows_v, hist_v,
             sem):
    wid = lax.axis_index("s") * NC + lax.axis_index("c")
    pltpu.sync_copy(idx_hbm.at[pl.ds(wid * N_CHUNK, N_CHUNK)], idx_v)
    copies = []
    for j in range(N_CHUNK):
        copies.append(pltpu.async_copy(
            table_hbm.at[idx_v.at[j]],
            rows_v.at[pl.ds(j * IDX_CHUNK, IDX_CHUNK)],
            sem))

    zeros16 = jnp.zeros((16,), jnp.int32)
    for h in range(N_EMB // 16):
        hist_v[pl.ds(h * 16, 16)] = zeros16

    for j in range(N_CHUNK):
        for c in range(IDX_CHUNK // 16):
            v = idx_v[j, pl.ds(c * 16, 16)]
            cnt, last = plsc.scan_count(v)
            plsc.addupdate_scatter(hist_v, [v], cnt, mask=last)

    for c in copies:
        c.wait()
    pltpu.sync_copy(rows_v, out_hbm.at[pl.ds(wid * TOK_PER_W, TOK_PER_W)])
    pltpu.sync_copy(hist_v, hist_hbm.at[wid])


@functools.cache
def _sc_gather():
    return pl.kernel(
        _sc_body,
        out_type=[
            jax.ShapeDtypeStruct((N_TOK, D), jnp.float32),
            jax.ShapeDtypeStruct((NW, N_EMB), jnp.int32),
        ],
        mesh=plsc.VectorSubcoreMesh(core_axis_name="c", subcore_axis_name="s",
                                    num_cores=NC, num_subcores=NS),
        scratch_types=[
            pltpu.VMEM((N_CHUNK, IDX_CHUNK), jnp.int32),
            pltpu.VMEM((TOK_PER_W, D), jnp.float32),
            pltpu.VMEM((N_EMB,), jnp.int32),
            pltpu.SemaphoreType.DMA,
        ],
        compiler_params=pltpu.CompilerParams(use_tc_tiling_on_sc=False,
                                             needs_layout_passes=False),
    )


def _ppl_body(hist_ref, ppl_ref):
    counts = jnp.sum(hist_ref[...].astype(jnp.float32), axis=0,
                     keepdims=True)
    p = counts * (1.0 / N_TOK)
    ent = jnp.sum(p * jnp.log(p + 1e-10))
    ppl_ref[...] = jnp.full((1, 1), jnp.exp(-ent), jnp.float32)


def kernel(inputs, codebook):
    flat = inputs.reshape(N_TOK, D)
    idx, loss = pl.pallas_call(
        _tc_body,
        grid=(GRID,),
        in_specs=[
            pl.BlockSpec((BLK, D), lambda i: (i, 0)),
            pl.BlockSpec((D, N_EMB), lambda i: (0, 0)),
        ],
        out_specs=[
            pl.BlockSpec((BLK, 1), lambda i: (i, 0)),
            pl.BlockSpec((1, 1), lambda i: (0, 0)),
        ],
        out_shape=[
            jax.ShapeDtypeStruct((N_TOK, 1), jnp.int32),
            jax.ShapeDtypeStruct((1, 1), jnp.float32),
        ],
        scratch_shapes=[
            pltpu.SMEM((1, 1), jnp.float32),
        ],
    )(flat, codebook)

    idx2 = idx.reshape(NW * N_CHUNK, IDX_CHUNK)
    qflat, hist = _sc_gather()(idx2, codebook.T)
    quantized = qflat.reshape(B_IN, HW_IN, D)

    ppl = pl.pallas_call(
        _ppl_body,
        out_shape=jax.ShapeDtypeStruct((1, 1), jnp.float32),
    )(hist)
    return quantized, loss[0, 0], ppl[0, 0]


# speedup vs baseline: 1.0642x; 1.0038x over previous
import functools

import jax
import jax.numpy as jnp
from jax import lax
from jax.experimental import pallas as pl
from jax.experimental.pallas import tpu as pltpu
from jax.experimental.pallas import tpu_sc as plsc

N_EMB = 1024
D = 64
B_IN = 16
HW_IN = 1024
N_TOK = B_IN * HW_IN
BLK = 4096
GRID = N_TOK // BLK

NC, NS = 2, 16
NW = NC * NS
TOK_PER_W = N_TOK // NW
IDX_CHUNK = 128
N_CHUNK = TOK_PER_W // IDX_CHUNK


def _tc_body(x_ref, cb_ref, idx_ref, loss_ref, acc_ref):
    i = pl.program_id(0)
    x = x_ref[...]
    cb = cb_ref[...]
    xn = jnp.sum(x * x, axis=1, keepdims=True)
    cn = jnp.sum(cb * cb, axis=0, keepdims=True)
    mm2 = jnp.dot(x * -2.0, cb, preferred_element_type=jnp.float32)
    d = (xn + cn) + mm2
    m = jnp.min(d, axis=1, keepdims=True)
    iota = lax.broadcasted_iota(jnp.int32, (BLK, N_EMB), 1)
    idx = jnp.min(jnp.where(d == m, iota, N_EMB), axis=1, keepdims=True)
    idx_ref[...] = idx

    @pl.when(i == 0)
    def _init():
        acc_ref[0, 0] = 0.0

    acc_ref[0, 0] += jnp.sum(m)
    loss_ref[...] = jnp.full((1, 1), acc_ref[0, 0] * (1.0 / (N_TOK * D)),
                             jnp.float32)


def _sc_body(idx_hbm, table_hbm, out_hbm, hist_hbm, idx_v, rows_v, hist_v,
             sem):
    wid = lax.axis_index("s") * NC + lax.axis_index("c")
    pltpu.sync_copy(idx_hbm.at[pl.ds(wid * N_CHUNK, N_CHUNK)], idx_v)
    copies = []
    for j in range(N_CHUNK):
        copies.append(pltpu.async_copy(
            table_hbm.at[idx_v.at[j]],
            rows_v.at[pl.ds(j * IDX_CHUNK, IDX_CHUNK)],
            sem))

    zeros16 = jnp.zeros((16,), jnp.int32)
    for h in range(N_EMB // 16):
        hist_v[pl.ds(h * 16, 16)] = zeros16

    for j in range(N_CHUNK):
        for c in range(IDX_CHUNK // 16):
            v = idx_v[j, pl.ds(c * 16, 16)]
            cnt, last = plsc.scan_count(v)
            plsc.addupdate_scatter(hist_v, [v], cnt, mask=last)

    for c in copies:
        c.wait()
    pltpu.sync_copy(rows_v, out_hbm.at[pl.ds(wid * TOK_PER_W, TOK_PER_W)])
    pltpu.sync_copy(hist_v, hist_hbm.at[wid])


@functools.cache
def _sc_gather():
    return pl.kernel(
        _sc_body,
        out_type=[
            jax.ShapeDtypeStruct((N_TOK, D), jnp.float32),
            jax.ShapeDtypeStruct((NW, N_EMB), jnp.int32),
        ],
        mesh=plsc.VectorSubcoreMesh(core_axis_name="c", subcore_axis_name="s",
                                    num_cores=NC, num_subcores=NS),
        scratch_types=[
            pltpu.VMEM((N_CHUNK, IDX_CHUNK), jnp.int32),
            pltpu.VMEM((TOK_PER_W, D), jnp.float32),
            pltpu.VMEM((N_EMB,), jnp.int32),
            pltpu.SemaphoreType.DMA,
        ],
        compiler_params=pltpu.CompilerParams(use_tc_tiling_on_sc=False,
                                             needs_layout_passes=False),
    )


def _ppl_body(hist_ref, ppl_ref):
    counts = jnp.sum(hist_ref[...].astype(jnp.float32), axis=0,
                     keepdims=True)
    p = counts * (1.0 / N_TOK)
    ent = jnp.sum(p * jnp.log(p + 1e-10))
    ppl_ref[...] = jnp.full((1, 1), jnp.exp(-ent), jnp.float32)


def kernel(inputs, codebook):
    flat = inputs.reshape(N_TOK, D)
    idx, loss = pl.pallas_call(
        _tc_body,
        grid=(GRID,),
        in_specs=[
            pl.BlockSpec((BLK, D), lambda i: (i, 0)),
            pl.BlockSpec((D, N_EMB), lambda i: (0, 0)),
        ],
        out_specs=[
            pl.BlockSpec((BLK, 1), lambda i: (i, 0)),
            pl.BlockSpec((1, 1), lambda i: (0, 0)),
        ],
        out_shape=[
            jax.ShapeDtypeStruct((N_TOK, 1), jnp.int32),
            jax.ShapeDtypeStruct((1, 1), jnp.float32),
        ],
        scratch_shapes=[
            pltpu.SMEM((1, 1), jnp.float32),
        ],
    )(flat, codebook)

    idx2 = idx.reshape(NW * N_CHUNK, IDX_CHUNK)
    qflat, hist = _sc_gather()(idx2, codebook.T)
    quantized = qflat.reshape(B_IN, HW_IN, D)

    ppl = pl.pallas_call(
        _ppl_body,
        out_shape=jax.ShapeDtypeStruct((1, 1), jnp.float32),
    )(hist)
    return quantized, loss[0, 0], ppl[0, 0]


# speedup vs baseline: 1.0706x; 1.0060x over previous
import functools

import jax
import jax.numpy as jnp
from jax import lax
from jax.experimental import pallas as pl
from jax.experimental.pallas import tpu as pltpu
from jax.experimental.pallas import tpu_sc as plsc

N_EMB = 1024
D = 64
B_IN = 16
HW_IN = 1024
N_TOK = B_IN * HW_IN
BLK = 4096
GRID = N_TOK // BLK

NC, NS = 2, 16
NW = NC * NS
TOK_PER_W = N_TOK // NW
IDX_CHUNK = 128
N_CHUNK = TOK_PER_W // IDX_CHUNK


def _tc_body(x_ref, cb_ref, idx_ref, loss_ref, acc_ref):
    i = pl.program_id(0)
    x = x_ref[...]
    cb = cb_ref[...]
    xn = jnp.sum(x * x, axis=1, keepdims=True)
    cn = jnp.sum(cb * cb, axis=0, keepdims=True)
    mm2 = jnp.dot(x * -2.0, cb, preferred_element_type=jnp.float32)
    d = (xn + cn) + mm2
    m = jnp.min(d, axis=1, keepdims=True)
    iota = lax.broadcasted_iota(jnp.int32, (BLK, N_EMB), 1)
    idx = jnp.min(jnp.where(d == m, iota, N_EMB), axis=1, keepdims=True)
    idx_ref[...] = idx.astype(jnp.int16)

    @pl.when(i == 0)
    def _init():
        acc_ref[0, 0] = 0.0

    acc_ref[0, 0] += jnp.sum(m)
    loss_ref[...] = jnp.full((1, 1), acc_ref[0, 0] * (1.0 / (N_TOK * D)),
                             jnp.float32)


def _sc_body(idx_hbm, table_hbm, out_hbm, hist_hbm, idx_v, rows_v, hist_v,
             sem):
    wid = lax.axis_index("s") * NC + lax.axis_index("c")
    pltpu.sync_copy(idx_hbm.at[pl.ds(wid * N_CHUNK, N_CHUNK)], idx_v)
    copies = []
    for j in range(N_CHUNK):
        copies.append(pltpu.async_copy(
            table_hbm.at[idx_v.at[j]],
            rows_v.at[pl.ds(j * IDX_CHUNK, IDX_CHUNK)],
            sem))

    zeros16 = jnp.zeros((16,), jnp.int32)
    for h in range(N_EMB // 16):
        hist_v[pl.ds(h * 16, 16)] = zeros16

    for j in range(N_CHUNK):
        for c in range(IDX_CHUNK // 16):
            v = idx_v[j, pl.ds(c * 16, 16)]
            cnt, last = plsc.scan_count(v)
            plsc.addupdate_scatter(hist_v, [v], cnt, mask=last)

    for c in copies:
        c.wait()
    pltpu.sync_copy(rows_v, out_hbm.at[pl.ds(wid * TOK_PER_W, TOK_PER_W)])
    pltpu.sync_copy(hist_v, hist_hbm.at[wid])


@functools.cache
def _sc_gather():
    return pl.kernel(
        _sc_body,
        out_type=[
            jax.ShapeDtypeStruct((N_TOK, D), jnp.float32),
            jax.ShapeDtypeStruct((NW, N_EMB), jnp.int32),
        ],
        mesh=plsc.VectorSubcoreMesh(core_axis_name="c", subcore_axis_name="s",
                                    num_cores=NC, num_subcores=NS),
        scratch_types=[
            pltpu.VMEM((N_CHUNK, IDX_CHUNK), jnp.int32),
            pltpu.VMEM((TOK_PER_W, D), jnp.float32),
            pltpu.VMEM((N_EMB,), jnp.int32),
            pltpu.SemaphoreType.DMA,
        ],
        compiler_params=pltpu.CompilerParams(use_tc_tiling_on_sc=False,
                                             needs_layout_passes=False),
    )


def _ppl_body(hist_ref, ppl_ref):
    counts = jnp.sum(hist_ref[...].astype(jnp.float32), axis=0,
                     keepdims=True)
    p = counts * (1.0 / N_TOK)
    ent = jnp.sum(p * jnp.log(p + 1e-10))
    ppl_ref[...] = jnp.full((1, 1), jnp.exp(-ent), jnp.float32)


def kernel(inputs, codebook):
    flat = inputs.reshape(N_TOK, D)
    idx, loss = pl.pallas_call(
        _tc_body,
        grid=(GRID,),
        in_specs=[
            pl.BlockSpec((BLK, D), lambda i: (i, 0)),
            pl.BlockSpec((D, N_EMB), lambda i: (0, 0)),
        ],
        out_specs=[
            pl.BlockSpec((BLK, 1), lambda i: (i, 0)),
            pl.BlockSpec((1, 1), lambda i: (0, 0)),
        ],
        out_shape=[
            jax.ShapeDtypeStruct((N_TOK, 1), jnp.int16),
            jax.ShapeDtypeStruct((1, 1), jnp.float32),
        ],
        scratch_shapes=[
            pltpu.SMEM((1, 1), jnp.float32),
        ],
    )(flat, codebook)

    idx2 = idx.astype(jnp.int32).reshape(NW * N_CHUNK, IDX_CHUNK)
    qflat, hist = _sc_gather()(idx2, codebook.T)
    quantized = qflat.reshape(B_IN, HW_IN, D)

    ppl = pl.pallas_call(
        _ppl_body,
        out_shape=jax.ShapeDtypeStruct((1, 1), jnp.float32),
    )(hist)
    return quantized, loss[0, 0], ppl[0, 0]


# speedup vs baseline: 1.0887x; 1.0169x over previous
import functools

import jax
import jax.numpy as jnp
from jax import lax
from jax.experimental import pallas as pl
from jax.experimental.pallas import tpu as pltpu
from jax.experimental.pallas import tpu_sc as plsc

N_EMB = 1024
D = 64
B_IN = 16
HW_IN = 1024
N_TOK = B_IN * HW_IN
BLK = 4096
GRID = N_TOK // BLK

NC, NS = 2, 16
NW = NC * NS
TOK_PER_W = N_TOK // NW
IDX_CHUNK = 128
N_CHUNK = TOK_PER_W // IDX_CHUNK


def _tc_body(x_ref, cb_ref, idx_ref, loss_ref, acc_ref):
    i = pl.program_id(0)
    x = x_ref[...]
    cb = cb_ref[...]
    xn = jnp.sum(x * x, axis=1, keepdims=True)
    cn = jnp.sum(cb * cb, axis=0, keepdims=True)
    mm2 = jnp.dot(x * -2.0, cb, preferred_element_type=jnp.float32)
    d = (xn + cn) + mm2
    m = jnp.min(d, axis=1, keepdims=True)
    iota = lax.broadcasted_iota(jnp.int32, (BLK, N_EMB), 1)
    idx = jnp.min(jnp.where(d == m, iota, N_EMB), axis=1, keepdims=True)
    idx_ref[...] = idx.astype(jnp.int16)

    @pl.when(i == 0)
    def _init():
        acc_ref[0, 0] = 0.0

    acc_ref[0, 0] += jnp.sum(m)
    loss_ref[...] = jnp.full((1, 1), acc_ref[0, 0] * (1.0 / (N_TOK * D)),
                             jnp.float32)


def _sc_body(idx_hbm, table_hbm, out_hbm, hist_hbm, idx_v, rows_v, hist_v,
             sem):
    wid = lax.axis_index("s") * NC + lax.axis_index("c")
    pltpu.sync_copy(idx_hbm.at[pl.ds(wid * N_CHUNK, N_CHUNK)], idx_v)
    copies = []
    for j in range(N_CHUNK):
        copies.append(pltpu.async_copy(
            table_hbm.at[idx_v.at[j]],
            rows_v.at[pl.ds(j * IDX_CHUNK, IDX_CHUNK)],
            sem))

    zeros16 = jnp.zeros((16,), jnp.int32)
    for h in range(N_EMB // 16):
        hist_v[pl.ds(h * 16, 16)] = zeros16

    for j in range(N_CHUNK):
        for c in range(IDX_CHUNK // 16):
            v = idx_v[j, pl.ds(c * 16, 16)]
            cnt, last = plsc.scan_count(v)
            plsc.addupdate_scatter(hist_v, [v], cnt, mask=last)

    for c in copies:
        c.wait()
    pltpu.sync_copy(rows_v, out_hbm.at[pl.ds(wid * TOK_PER_W, TOK_PER_W)])
    pltpu.sync_copy(hist_v, hist_hbm.at[wid])


@functools.cache
def _sc_gather():
    return pl.kernel(
        _sc_body,
        out_type=[
            jax.ShapeDtypeStruct((N_TOK, D), jnp.float32),
            jax.ShapeDtypeStruct((NW, N_EMB), jnp.int32),
        ],
        mesh=plsc.VectorSubcoreMesh(core_axis_name="c", subcore_axis_name="s",
                                    num_cores=NC, num_subcores=NS),
        scratch_types=[
            pltpu.VMEM((N_CHUNK, IDX_CHUNK), jnp.int32),
            pltpu.VMEM((TOK_PER_W, D), jnp.float32),
            pltpu.VMEM((N_EMB,), jnp.int32),
            pltpu.SemaphoreType.DMA,
        ],
        compiler_params=pltpu.CompilerParams(use_tc_tiling_on_sc=False,
                                             needs_layout_passes=False),
    )


def _ppl_body(hist_ref, ppl_ref):
    counts = jnp.sum(hist_ref[...].astype(jnp.float32), axis=0,
                     keepdims=True)
    p = counts * (1.0 / N_TOK)
    ent = jnp.sum(p * jnp.log(p + 1e-10))
    ppl_ref[...] = jnp.full((1, 1), jnp.exp(-ent), jnp.float32)


def kernel(inputs, codebook):
    flat = inputs.reshape(N_TOK, D)
    idx, loss = pl.pallas_call(
        _tc_body,
        grid=(GRID,),
        in_specs=[
            pl.BlockSpec((BLK, D), lambda i: (i, 0)),
            pl.BlockSpec((D, N_EMB), lambda i: (0, 0)),
        ],
        out_specs=[
            pl.BlockSpec((BLK, 1), lambda i: (i, 0)),
            pl.BlockSpec((1, 1), lambda i: (0, 0)),
        ],
        out_shape=[
            jax.ShapeDtypeStruct((N_TOK, 1), jnp.int16),
            jax.ShapeDtypeStruct((1, 1), jnp.float32),
        ],
        scratch_shapes=[
            pltpu.SMEM((1, 1), jnp.float32),
        ],
        compiler_params=pltpu.CompilerParams(
            vmem_limit_bytes=120 * 1024 * 1024),
    )(flat, codebook)

    idx2 = idx.astype(jnp.int32).reshape(NW * N_CHUNK, IDX_CHUNK)
    qflat, hist = _sc_gather()(idx2, codebook.T)
    quantized = qflat.reshape(B_IN, HW_IN, D)

    ppl = pl.pallas_call(
        _ppl_body,
        out_shape=jax.ShapeDtypeStruct((1, 1), jnp.float32),
    )(hist)
    return quantized, loss[0, 0], ppl[0, 0]
